# Initial kernel scaffold; baseline (speedup 1.0000x reference)
import functools

import jax
import jax.numpy as jnp
from jax import lax
from jax.experimental import pallas as pl
from jax.experimental.pallas import tpu as pltpu
from jax.experimental.pallas import tpu_sc as plsc

E = 800000
N_NEI = 4
IN_C = 16
MID = 32
POOL = 32
CAT = 40

NW = 32
ROWS_W = E // NW
R = 200
NCHUNK = ROWS_W // R

BM = 8000


def _proj_body(x_ref, w_ref, b_ref, s_ref, y0_ref, y1_ref, y2_ref, y3_ref):
    h = jnp.dot(x_ref[...], w_ref[...], preferred_element_type=jnp.float32)
    h = h + b_ref[...]
    s_ref[...] = h[:, 0:32]
    y0_ref[...] = h[:, 32:64]
    y1_ref[...] = h[:, 64:96]
    y2_ref[...] = h[:, 96:128]
    y3_ref[...] = h[:, 128:160]


def _tc_projections(x, wall, ball):
    c = x.shape[1]
    outs = pl.pallas_call(
        _proj_body,
        grid=(E // BM,),
        in_specs=[
            pl.BlockSpec((BM, c), lambda i: (i, 0)),
            pl.BlockSpec((c, 160), lambda i: (0, 0)),
            pl.BlockSpec((1, 160), lambda i: (0, 0)),
        ],
        out_specs=tuple(pl.BlockSpec((BM, MID), lambda i: (i, 0))
                        for _ in range(5)),
        out_shape=tuple(jax.ShapeDtypeStruct((E, MID), jnp.float32)
                        for _ in range(5)),
    )(x, wall, ball)
    return outs


def _make_sc_combine(has_skip):
    mesh = plsc.VectorSubcoreMesh(core_axis_name="c", subcore_axis_name="s")

    def body(*refs):
        if has_skip:
            (s_hbm, y0h, y1h, y2h, y3h, idx_hbm, skip_hbm, out_hbm,
             i0, i1, i2, i3, sb, g0, g1, g2, g3, kb, ob, sem) = refs
        else:
            (s_hbm, y0h, y1h, y2h, y3h, idx_hbm, out_hbm,
             i0, i1, i2, i3, sb, g0, g1, g2, g3, ob, sem) = refs
            kb = None
        wid = lax.axis_index("s") * 2 + lax.axis_index("c")
        base = wid * ROWS_W

        def chunk(k, carry):
            off = base + k * R
            pltpu.sync_copy(idx_hbm.at[0, pl.ds(off, R)], i0)
            pltpu.sync_copy(idx_hbm.at[1, pl.ds(off, R)], i1)
            pltpu.sync_copy(idx_hbm.at[2, pl.ds(off, R)], i2)
            pltpu.sync_copy(idx_hbm.at[3, pl.ds(off, R)], i3)
            c0 = pltpu.async_copy(y0h.at[i0], g0, sem)
            c1 = pltpu.async_copy(y1h.at[i1], g1, sem)
            c2 = pltpu.async_copy(y2h.at[i2], g2, sem)
            c3 = pltpu.async_copy(y3h.at[i3], g3, sem)
            pltpu.sync_copy(s_hbm.at[pl.ds(off, R)], sb)
            if has_skip:
                pltpu.sync_copy(skip_hbm.at[pl.ds(off, R)], kb)
            c0.wait()
            c1.wait()
            c2.wait()
            c3.wait()

            def row(r, rcarry):
                for c in (0, 16):
                    sl = pl.ds(c, 16)
                    v = (sb[r, sl] + g0[r, sl] + g1[r, sl]
                         + g2[r, sl] + g3[r, sl])
                    v = jnp.maximum(v, 0.0)
                    if has_skip:
                        v = jnp.maximum(v + kb[r, sl], 0.0)
                    ob[r, sl] = v
                return rcarry

            lax.fori_loop(0, R, row, 0)
            pltpu.sync_copy(ob, out_hbm.at[pl.ds(off, R)])
            return carry

        lax.fori_loop(0, NCHUNK, chunk, 0)

    scratch = [pltpu.VMEM((R,), jnp.int32)] * 4
    scratch += [pltpu.VMEM((R, MID), jnp.float32)] * (6 if has_skip else 5)
    scratch += [pltpu.SemaphoreType.DMA]

    return functools.partial(
        pl.kernel,
        mesh=mesh,
        out_type=jax.ShapeDtypeStruct((E, MID), jnp.float32),
        scratch_types=scratch,
    )(body)


_sc_combine = _make_sc_combine(False)
_sc_combine_skip = _make_sc_combine(True)


def _pool_body(h_ref, wf_ref, bf_ref, o_ref):
    p = pl.program_id(0)
    m = jnp.mean(h_ref[...], axis=0).reshape(1, MID)
    part = jnp.dot(m, wf_ref[...], preferred_element_type=jnp.float32)

    @pl.when(p == 0)
    def _():
        o_ref[...] = bf_ref[...]

    o_ref[...] += part


def _pool_fc(h, wf, bf):
    seg = E // POOL
    out = pl.pallas_call(
        _pool_body,
        grid=(POOL,),
        in_specs=[
            pl.BlockSpec((seg, MID), lambda p: (p, 0)),
            pl.BlockSpec((MID, CAT), lambda p: (p, 0)),
            pl.BlockSpec((1, CAT), lambda p: (0, 0)),
        ],
        out_specs=pl.BlockSpec((1, CAT), lambda p: (0, 0)),
        out_shape=jax.ShapeDtypeStruct((1, CAT), jnp.float32),
    )(h, wf, bf.reshape(1, CAT))
    return out.reshape(CAT)


def _prep(w, b, c):
    wall = w.reshape(1 + N_NEI, c, MID).transpose(1, 0, 2).reshape(c, 160)
    ball = jnp.concatenate([b, jnp.zeros((4 * MID,), jnp.float32)]).reshape(1, 160)
    return wall, ball


def _conv(x, idx_t, wall, ball, skip=None):
    s, y0, y1, y2, y3 = _tc_projections(x, wall, ball)
    if skip is None:
        return _sc_combine(s, y0, y1, y2, y3, idx_t)
    return _sc_combine_skip(s, y0, y1, y2, y3, idx_t, skip)


def kernel(x, half_edges, W0, b0, W11, b11, W12, b12, W21, b21, W22, b22, Wf, bf):
    idx_t = half_edges.T.reshape(N_NEI, E)

    h = _conv(x, idx_t, *_prep(W0, b0, IN_C))
    for (wa, ba, wb, bb) in ((W11, b11, W12, b12), (W21, b21, W22, b22)):
        y = _conv(h, idx_t, *_prep(wa, ba, MID))
        h = _conv(y, idx_t, *_prep(wb, bb, MID), skip=h)
    return _pool_fc(h, Wf, bf)


# speedup vs baseline: 8.7434x; 8.7434x over previous
import functools

import jax
import jax.numpy as jnp
from jax import lax
from jax.experimental import pallas as pl
from jax.experimental.pallas import tpu as pltpu
from jax.experimental.pallas import tpu_sc as plsc

E = 800000
N_NEI = 4
IN_C = 16
MID = 32
POOL = 32
CAT = 40

NW = 32
ROWS_W = E // NW
R = 200
NCHUNK = ROWS_W // R

BM = 8000


def _proj_body(x_ref, w_ref, b_ref, s_ref, y0_ref, y1_ref, y2_ref, y3_ref):
    h = jnp.dot(x_ref[...], w_ref[...], preferred_element_type=jnp.float32)
    h = h + b_ref[...]
    s_ref[...] = h[:, 0:32]
    y0_ref[...] = h[:, 32:64]
    y1_ref[...] = h[:, 64:96]
    y2_ref[...] = h[:, 96:128]
    y3_ref[...] = h[:, 128:160]


def _tc_projections(x, wall, ball):
    c = x.shape[1]
    outs = pl.pallas_call(
        _proj_body,
        grid=(E // BM,),
        in_specs=[
            pl.BlockSpec((BM, c), lambda i: (i, 0)),
            pl.BlockSpec((c, 160), lambda i: (0, 0)),
            pl.BlockSpec((1, 160), lambda i: (0, 0)),
        ],
        out_specs=tuple(pl.BlockSpec((BM, MID), lambda i: (i, 0))
                        for _ in range(5)),
        out_shape=tuple(jax.ShapeDtypeStruct((E, MID), jnp.float32)
                        for _ in range(5)),
    )(x, wall, ball)
    return outs


def _make_sc_combine(has_skip):
    mesh = plsc.VectorSubcoreMesh(core_axis_name="c", subcore_axis_name="s")

    def body(*refs):
        if has_skip:
            (s_hbm, y0h, y1h, y2h, y3h, ix0, ix1, ix2, ix3, skip_hbm, out_hbm,
             i0, i1, i2, i3, sb, g0, g1, g2, g3, kb, ob, sem) = refs
        else:
            (s_hbm, y0h, y1h, y2h, y3h, ix0, ix1, ix2, ix3, out_hbm,
             i0, i1, i2, i3, sb, g0, g1, g2, g3, ob, sem) = refs
            kb = None
        wid = lax.axis_index("s") * 2 + lax.axis_index("c")
        base = wid * ROWS_W

        def chunk(k, carry):
            off = base + k * R
            pltpu.sync_copy(ix0.at[pl.ds(off, R)], i0)
            pltpu.sync_copy(ix1.at[pl.ds(off, R)], i1)
            pltpu.sync_copy(ix2.at[pl.ds(off, R)], i2)
            pltpu.sync_copy(ix3.at[pl.ds(off, R)], i3)
            c0 = pltpu.async_copy(y0h.at[i0], g0, sem)
            c1 = pltpu.async_copy(y1h.at[i1], g1, sem)
            c2 = pltpu.async_copy(y2h.at[i2], g2, sem)
            c3 = pltpu.async_copy(y3h.at[i3], g3, sem)
            pltpu.sync_copy(s_hbm.at[pl.ds(off, R)], sb)
            if has_skip:
                pltpu.sync_copy(skip_hbm.at[pl.ds(off, R)], kb)
            c0.wait()
            c1.wait()
            c2.wait()
            c3.wait()

            def row(r, rcarry):
                for c in (0, 16):
                    sl = pl.ds(c, 16)
                    v = (sb[r, sl] + g0[r, sl] + g1[r, sl]
                         + g2[r, sl] + g3[r, sl])
                    v = jnp.maximum(v, 0.0)
                    if has_skip:
                        v = jnp.maximum(v + kb[r, sl], 0.0)
                    ob[r, sl] = v
                return rcarry

            lax.fori_loop(0, R, row, 0)
            pltpu.sync_copy(ob, out_hbm.at[pl.ds(off, R)])
            return carry

        lax.fori_loop(0, NCHUNK, chunk, 0)

    scratch = [pltpu.VMEM((R,), jnp.int32)] * 4
    scratch += [pltpu.VMEM((R, MID), jnp.float32)] * (7 if has_skip else 6)
    scratch += [pltpu.SemaphoreType.DMA]

    return functools.partial(
        pl.kernel,
        mesh=mesh,
        out_type=jax.ShapeDtypeStruct((E, MID), jnp.float32),
        scratch_types=scratch,
        compiler_params=pltpu.CompilerParams(use_tc_tiling_on_sc=False),
    )(body)


_sc_combine = _make_sc_combine(False)
_sc_combine_skip = _make_sc_combine(True)


def _pool_body(h_ref, wf_ref, bf_ref, o_ref):
    p = pl.program_id(0)
    m = jnp.mean(h_ref[...], axis=0).reshape(1, MID)
    part = jnp.dot(m, wf_ref[...], preferred_element_type=jnp.float32)

    @pl.when(p == 0)
    def _():
        o_ref[...] = bf_ref[...]

    o_ref[...] += part


def _pool_fc(h, wf, bf):
    seg = E // POOL
    out = pl.pallas_call(
        _pool_body,
        grid=(POOL,),
        in_specs=[
            pl.BlockSpec((seg, MID), lambda p: (p, 0)),
            pl.BlockSpec((MID, CAT), lambda p: (p, 0)),
            pl.BlockSpec((1, CAT), lambda p: (0, 0)),
        ],
        out_specs=pl.BlockSpec((1, CAT), lambda p: (0, 0)),
        out_shape=jax.ShapeDtypeStruct((1, CAT), jnp.float32),
    )(h, wf, bf.reshape(1, CAT))
    return out.reshape(CAT)


def _prep(w, b, c):
    wall = w.reshape(1 + N_NEI, c, MID).transpose(1, 0, 2).reshape(c, 160)
    ball = jnp.concatenate([b, jnp.zeros((4 * MID,), jnp.float32)]).reshape(1, 160)
    return wall, ball


def _conv(x, idx_t, wall, ball, skip=None):
    s, y0, y1, y2, y3 = _tc_projections(x, wall, ball)
    if skip is None:
        return _sc_combine(s, y0, y1, y2, y3, *idx_t)
    return _sc_combine_skip(s, y0, y1, y2, y3, *idx_t, skip)


def kernel(x, half_edges, W0, b0, W11, b11, W12, b12, W21, b21, W22, b22, Wf, bf):
    idx_t = [half_edges[:, j].reshape(E) for j in range(N_NEI)]

    h = _conv(x, idx_t, *_prep(W0, b0, IN_C))
    for (wa, ba, wb, bb) in ((W11, b11, W12, b12), (W21, b21, W22, b22)):
        y = _conv(h, idx_t, *_prep(wa, ba, MID))
        h = _conv(y, idx_t, *_prep(wb, bb, MID), skip=h)
    return _pool_fc(h, Wf, bf)
